# baseline (speedup 1.0000x reference)
import functools

import jax
import jax.numpy as jnp
from jax import lax
from jax.experimental import pallas as pl
from jax.experimental.pallas import tpu as pltpu
from jax.experimental.pallas import tpu_sc as plsc

B, C, H, W = 4, 96, 224, 224
Hp, Wp = H + 2, W + 2
BHW = B * H * W
NROWS = B * Hp * Wp
NW = 32
PPW = BHW // NW
SUB = 128
NSUB = PPW // SUB
LANES = 16

_GATHER_DNUMS = lax.GatherDimensionNumbers(
    offset_dims=(), collapsed_slice_dims=(0,), start_index_map=(0,))


def _pad_body(x_ref, o_ref):
    x = x_ref[0]
    xr = jnp.concatenate([x[:, :, :1], x, x[:, :, W - 1:]], axis=2)
    o_ref[0] = jnp.concatenate([xr[:, :1, :], xr, xr[:, H - 1:, :]], axis=1)


def _prep_body(d_ref, i00_ref, i10_ref, i01_ref, i11_ref, dh_ref, dw_ref):
    b = pl.program_id(0)
    rc = pl.program_id(1)
    rows = d_ref.shape[2]
    dH = d_ref[0, 0]
    dW = d_ref[0, 1]
    hg = (lax.broadcasted_iota(jnp.int32, (rows, W), 0)
          + rc * rows).astype(jnp.float32)
    wg = lax.broadcasted_iota(jnp.int32, (rows, W), 1).astype(jnp.float32)
    H_up = hg + dH + 1.0
    W_up = wg + dW + 1.0
    hf = jnp.floor(H_up)
    wf = jnp.floor(W_up)
    hfp = jnp.clip(hf, 0.0, Hp - 1.0)
    hcp = jnp.clip(hf + 1.0, 0.0, Hp - 1.0)
    wfp = jnp.clip(wf, 0.0, Wp - 1.0)
    wcp = jnp.clip(wf + 1.0, 0.0, Wp - 1.0)
    dh_ref[0] = hcp - H_up
    dw_ref[0] = wcp - W_up
    h0 = hfp.astype(jnp.int32)
    h1 = hcp.astype(jnp.int32)
    w0 = wfp.astype(jnp.int32)
    w1 = wcp.astype(jnp.int32)
    basei = b * (Hp * Wp)
    r0 = basei + h0 * Wp
    r1 = basei + h1 * Wp
    i00_ref[0] = r0 + w0
    i10_ref[0] = r0 + w1
    i01_ref[0] = r1 + w0
    i11_ref[0] = r1 + w1


def _tout_body(v_ref, o_ref):
    v = v_ref[...]
    o_ref[0] = v.T.reshape(C, v.shape[0] // W, W)


def _sc_warp_body(tbl, i00, i10, i01, i11, dhw, dww, out,
                  i00v, i10v, i01v, i11v, dhv, dwv,
                  b00, b10, b01, b11, ov, s0, s1, s2, s3):
    cid = lax.axis_index("c")
    sid = lax.axis_index("s")
    wid = sid * 2 + cid
    base = wid * PPW

    def step(g, carry):
        off = base + g * SUB
        pltpu.sync_copy(i00.at[pl.ds(off, SUB)], i00v)
        pltpu.sync_copy(i10.at[pl.ds(off, SUB)], i10v)
        pltpu.sync_copy(i01.at[pl.ds(off, SUB)], i01v)
        pltpu.sync_copy(i11.at[pl.ds(off, SUB)], i11v)
        pltpu.sync_copy(dhw.at[pl.ds(off, SUB)], dhv)
        pltpu.sync_copy(dww.at[pl.ds(off, SUB)], dwv)
        c0 = pltpu.async_copy(tbl.at[i00v], b00, s0)
        c1 = pltpu.async_copy(tbl.at[i10v], b10, s1)
        c2 = pltpu.async_copy(tbl.at[i01v], b01, s2)
        c3 = pltpu.async_copy(tbl.at[i11v], b11, s3)
        c0.wait()
        c1.wait()
        c2.wait()
        c3.wait()

        def blend(gg, carry2):
            p0 = gg * LANES
            dh = dhv[pl.ds(p0, LANES)]
            dw = dwv[pl.ds(p0, LANES)]
            for j in range(LANES):
                p = p0 + j
                sel = jnp.full((LANES, 1), j, jnp.int32)
                dhp = lax.gather(dh, sel, _GATHER_DNUMS, (1,),
                                 mode=lax.GatherScatterMode.PROMISE_IN_BOUNDS)
                dwp = lax.gather(dw, sel, _GATHER_DNUMS, (1,),
                                 mode=lax.GatherScatterMode.PROMISE_IN_BOUNDS)
                w00 = dhp * dwp
                w10 = dhp - w00
                w01 = dwp - w00
                w11 = 1.0 - dhp - dwp + w00
                for k in range(C // LANES):
                    cs = pl.ds(k * LANES, LANES)
                    acc = (b00[p, cs] * w00 + b10[p, cs] * w10
                           + b01[p, cs] * w01 + b11[p, cs] * w11)
                    ov[p, cs] = acc
            return carry2

        lax.fori_loop(0, SUB // LANES, blend, 0)
        pltpu.sync_copy(ov, out.at[pl.ds(off, SUB)])
        return carry

    lax.fori_loop(0, NSUB, step, 0)


def kernel(input1, input2):
    padded = jnp.pad(input1, ((0, 0), (0, 0), (1, 1), (1, 1)), mode="edge")
    tbl = padded.reshape(NROWS, C)

    RB = 8
    hw_spec = pl.BlockSpec((1, RB, W), lambda b, r: (b, r, 0))
    i00, i10, i01, i11, dhw, dww = pl.pallas_call(
        _prep_body,
        grid=(B, H // RB),
        in_specs=[pl.BlockSpec((1, 2, RB, W), lambda b, r: (b, 0, r, 0))],
        out_specs=[hw_spec] * 6,
        out_shape=[jax.ShapeDtypeStruct((B, H, W), jnp.int32)] * 4
        + [jax.ShapeDtypeStruct((B, H, W), jnp.float32)] * 2,
    )(input2)

    mesh = plsc.VectorSubcoreMesh(core_axis_name="c", subcore_axis_name="s")
    vals = functools.partial(
        pl.kernel,
        out_type=jax.ShapeDtypeStruct((BHW, C), jnp.float32),
        mesh=mesh,
        compiler_params=pltpu.CompilerParams(use_tc_tiling_on_sc=False),
        scratch_types=[pltpu.VMEM((SUB,), jnp.int32)] * 4
        + [pltpu.VMEM((SUB,), jnp.float32)] * 2
        + [pltpu.VMEM((SUB, C), jnp.float32)] * 5
        + [pltpu.SemaphoreType.DMA] * 4,
    )(_sc_warp_body)(
        tbl,
        i00.reshape(BHW),
        i10.reshape(BHW),
        i01.reshape(BHW),
        i11.reshape(BHW),
        dhw.reshape(BHW),
        dww.reshape(BHW),
    )

    return vals.reshape(B, H, W, C).transpose(0, 3, 1, 2)


# speedup vs baseline: 1.0706x; 1.0706x over previous
import functools

import jax
import jax.numpy as jnp
from jax import lax
from jax.experimental import pallas as pl
from jax.experimental.pallas import tpu as pltpu
from jax.experimental.pallas import tpu_sc as plsc

B, C, H, W = 4, 96, 224, 224
Hp, Wp = H + 2, W + 2
BHW = B * H * W
NROWS = B * Hp * Wp
NW = 32
PPW = BHW // NW
SUB = 128
NSUB = PPW // SUB
LANES = 16

_GATHER_DNUMS = lax.GatherDimensionNumbers(
    offset_dims=(), collapsed_slice_dims=(0,), start_index_map=(0,))


def _pad_body(x_ref, o_ref):
    x = x_ref[0]
    xr = jnp.concatenate([x[:, :, :1], x, x[:, :, W - 1:]], axis=2)
    o_ref[0] = jnp.concatenate([xr[:, :1, :], xr, xr[:, H - 1:, :]], axis=1)


def _prep_body(d_ref, i00_ref, i10_ref, i01_ref, i11_ref, dh_ref, dw_ref):
    b = pl.program_id(0)
    rc = pl.program_id(1)
    rows = d_ref.shape[2]
    dH = d_ref[0, 0]
    dW = d_ref[0, 1]
    hg = (lax.broadcasted_iota(jnp.int32, (rows, W), 0)
          + rc * rows).astype(jnp.float32)
    wg = lax.broadcasted_iota(jnp.int32, (rows, W), 1).astype(jnp.float32)
    H_up = hg + dH + 1.0
    W_up = wg + dW + 1.0
    hf = jnp.floor(H_up)
    wf = jnp.floor(W_up)
    hfp = jnp.clip(hf, 0.0, Hp - 1.0)
    hcp = jnp.clip(hf + 1.0, 0.0, Hp - 1.0)
    wfp = jnp.clip(wf, 0.0, Wp - 1.0)
    wcp = jnp.clip(wf + 1.0, 0.0, Wp - 1.0)
    dh_ref[0] = hcp - H_up
    dw_ref[0] = wcp - W_up
    h0 = hfp.astype(jnp.int32)
    h1 = hcp.astype(jnp.int32)
    w0 = wfp.astype(jnp.int32)
    w1 = wcp.astype(jnp.int32)
    basei = b * (Hp * Wp)
    r0 = basei + h0 * Wp
    r1 = basei + h1 * Wp
    i00_ref[0] = r0 + w0
    i10_ref[0] = r0 + w1
    i01_ref[0] = r1 + w0
    i11_ref[0] = r1 + w1


def _tout_body(v_ref, o_ref):
    v = v_ref[...]
    o_ref[0] = v.T.reshape(C, v.shape[0] // W, W)


def _sc_warp_body(tbl, i00, i10, i01, i11, dhw, dww, out,
                  i00v, i10v, i01v, i11v, dhv, dwv,
                  b00, b10, b01, b11, ov, s0, s1, s2, s3):
    cid = lax.axis_index("c")
    sid = lax.axis_index("s")
    wid = sid * 2 + cid
    base = wid * PPW

    def step(g, carry):
        off = base + g * SUB
        pltpu.sync_copy(i00.at[pl.ds(off, SUB)], i00v)
        pltpu.sync_copy(i10.at[pl.ds(off, SUB)], i10v)
        pltpu.sync_copy(i01.at[pl.ds(off, SUB)], i01v)
        pltpu.sync_copy(i11.at[pl.ds(off, SUB)], i11v)
        pltpu.sync_copy(dhw.at[pl.ds(off, SUB)], dhv)
        pltpu.sync_copy(dww.at[pl.ds(off, SUB)], dwv)
        c0 = pltpu.async_copy(tbl.at[i00v], b00, s0)
        c1 = pltpu.async_copy(tbl.at[i10v], b10, s1)
        c2 = pltpu.async_copy(tbl.at[i01v], b01, s2)
        c3 = pltpu.async_copy(tbl.at[i11v], b11, s3)
        c0.wait()
        c1.wait()
        c2.wait()
        c3.wait()

        def blend(gg, carry2):
            p0 = gg * LANES
            dh = dhv[pl.ds(p0, LANES)]
            dw = dwv[pl.ds(p0, LANES)]
            for j in range(LANES):
                p = p0 + j
                sel = jnp.full((LANES, 1), j, jnp.int32)
                dhp = lax.gather(dh, sel, _GATHER_DNUMS, (1,),
                                 mode=lax.GatherScatterMode.PROMISE_IN_BOUNDS)
                dwp = lax.gather(dw, sel, _GATHER_DNUMS, (1,),
                                 mode=lax.GatherScatterMode.PROMISE_IN_BOUNDS)
                w00 = dhp * dwp
                w10 = dhp - w00
                w01 = dwp - w00
                w11 = 1.0 - dhp - dwp + w00
                for k in range(C // LANES):
                    cs = pl.ds(k * LANES, LANES)
                    acc = (b00[p, cs] * w00 + b10[p, cs] * w10
                           + b01[p, cs] * w01 + b11[p, cs] * w11)
                    ov[p, cs] = acc
            return carry2

        lax.fori_loop(0, SUB // LANES, blend, 0)
        pltpu.sync_copy(ov, out.at[pl.ds(off, SUB)])
        return carry

    lax.fori_loop(0, NSUB, step, 0)


def kernel(input1, input2):
    padded = pl.pallas_call(
        _pad_body,
        grid=(B, C // 8),
        in_specs=[pl.BlockSpec((1, 8, H, W), lambda b, c: (b, c, 0, 0))],
        out_specs=pl.BlockSpec((1, 8, Hp, Wp), lambda b, c: (b, c, 0, 0)),
        out_shape=jax.ShapeDtypeStruct((B, C, Hp, Wp), jnp.float32),
    )(input1)
    tbl = padded.reshape(NROWS, C)

    RB = 8
    hw_spec = pl.BlockSpec((1, RB, W), lambda b, r: (b, r, 0))
    i00, i10, i01, i11, dhw, dww = pl.pallas_call(
        _prep_body,
        grid=(B, H // RB),
        in_specs=[pl.BlockSpec((1, 2, RB, W), lambda b, r: (b, 0, r, 0))],
        out_specs=[hw_spec] * 6,
        out_shape=[jax.ShapeDtypeStruct((B, H, W), jnp.int32)] * 4
        + [jax.ShapeDtypeStruct((B, H, W), jnp.float32)] * 2,
    )(input2)

    mesh = plsc.VectorSubcoreMesh(core_axis_name="c", subcore_axis_name="s")
    vals = functools.partial(
        pl.kernel,
        out_type=jax.ShapeDtypeStruct((BHW, C), jnp.float32),
        mesh=mesh,
        compiler_params=pltpu.CompilerParams(use_tc_tiling_on_sc=False),
        scratch_types=[pltpu.VMEM((SUB,), jnp.int32)] * 4
        + [pltpu.VMEM((SUB,), jnp.float32)] * 2
        + [pltpu.VMEM((SUB, C), jnp.float32)] * 5
        + [pltpu.SemaphoreType.DMA] * 4,
    )(_sc_warp_body)(
        tbl,
        i00.reshape(BHW),
        i10.reshape(BHW),
        i01.reshape(BHW),
        i11.reshape(BHW),
        dhw.reshape(BHW),
        dww.reshape(BHW),
    )

    return vals.reshape(B, H, W, C).transpose(0, 3, 1, 2)


# speedup vs baseline: 1.2438x; 1.1618x over previous
import functools

import jax
import jax.numpy as jnp
from jax import lax
from jax.experimental import pallas as pl
from jax.experimental.pallas import tpu as pltpu
from jax.experimental.pallas import tpu_sc as plsc

B, C, H, W = 4, 96, 224, 224
Hp, Wp = H + 2, W + 2
BHW = B * H * W
NROWS = B * Hp * Wp
NW = 32
PPW = BHW // NW
SUB = 128
NSUB = PPW // SUB
LANES = 16

_GATHER_DNUMS = lax.GatherDimensionNumbers(
    offset_dims=(), collapsed_slice_dims=(0,), start_index_map=(0,))


def _pad_body(x_ref, o_ref):
    x = x_ref[0]
    xr = jnp.concatenate([x[:, :, :1], x, x[:, :, W - 1:]], axis=2)
    o_ref[0] = jnp.concatenate([xr[:, :1, :], xr, xr[:, H - 1:, :]], axis=1)


def _prep_body(d_ref, i00_ref, i10_ref, i01_ref, i11_ref, dh_ref, dw_ref):
    b = pl.program_id(0)
    rc = pl.program_id(1)
    rows = d_ref.shape[2]
    dH = d_ref[0, 0]
    dW = d_ref[0, 1]
    hg = (lax.broadcasted_iota(jnp.int32, (rows, W), 0)
          + rc * rows).astype(jnp.float32)
    wg = lax.broadcasted_iota(jnp.int32, (rows, W), 1).astype(jnp.float32)
    H_up = hg + dH + 1.0
    W_up = wg + dW + 1.0
    hf = jnp.floor(H_up)
    wf = jnp.floor(W_up)
    hfp = jnp.clip(hf, 0.0, Hp - 1.0)
    hcp = jnp.clip(hf + 1.0, 0.0, Hp - 1.0)
    wfp = jnp.clip(wf, 0.0, Wp - 1.0)
    wcp = jnp.clip(wf + 1.0, 0.0, Wp - 1.0)
    dh_ref[0] = hcp - H_up
    dw_ref[0] = wcp - W_up
    h0 = hfp.astype(jnp.int32)
    h1 = hcp.astype(jnp.int32)
    w0 = wfp.astype(jnp.int32)
    w1 = wcp.astype(jnp.int32)
    basei = b * (Hp * Wp)
    r0 = basei + h0 * Wp
    r1 = basei + h1 * Wp
    i00_ref[0] = r0 + w0
    i10_ref[0] = r0 + w1
    i01_ref[0] = r1 + w0
    i11_ref[0] = r1 + w1


def _tout_body(v_ref, o_ref):
    v = v_ref[...]
    o_ref[0] = v.T.reshape(C, v.shape[0] // W, W)


def _sc_warp_body(tbl, i00, i10, i01, i11, dhw, dww, out,
                  ia0, ia1, ia2, ia3, wa0, wa1,
                  ib0, ib1, ib2, ib3, wb0, wb1,
                  ba0, ba1, ba2, ba3, bb0, bb1, bb2, bb3, ov,
                  semIA, semIB, semGA, semGB):
    cid = lax.axis_index("c")
    sid = lax.axis_index("s")
    wid = sid * 2 + cid
    base = wid * PPW
    srcs = (i00, i10, i01, i11)
    bufsA = (ba0, ba1, ba2, ba3)
    bufsB = (bb0, bb1, bb2, bb3)
    iA = (ia0, ia1, ia2, ia3)
    iB = (ib0, ib1, ib2, ib3)

    def fire_chunk(g, ii, dd, bufs, semI, semG):
        off = base + g * SUB
        cs = [pltpu.async_copy(s.at[pl.ds(off, SUB)], d, semI)
              for s, d in zip(srcs, ii)]
        cs.append(pltpu.async_copy(dhw.at[pl.ds(off, SUB)], dd[0], semI))
        cs.append(pltpu.async_copy(dww.at[pl.ds(off, SUB)], dd[1], semI))
        for c in cs:
            c.wait()
        for d, buf in zip(ii, bufs):
            pltpu.async_copy(tbl.at[d], buf, semG)

    def wait_chunk(bufs, semG):
        for buf in bufs:
            pltpu.make_async_copy(tbl.at[pl.ds(0, SUB)], buf, semG).wait()

    def blend_write(g, bufs, dd):
        def blend(gg, carry2):
            p0 = gg * LANES
            dh = dd[0][pl.ds(p0, LANES)]
            dw = dd[1][pl.ds(p0, LANES)]
            for j in range(LANES):
                p = p0 + j
                sel = jnp.full((LANES, 1), j, jnp.int32)
                dhp = lax.gather(dh, sel, _GATHER_DNUMS, (1,),
                                 mode=lax.GatherScatterMode.PROMISE_IN_BOUNDS)
                dwp = lax.gather(dw, sel, _GATHER_DNUMS, (1,),
                                 mode=lax.GatherScatterMode.PROMISE_IN_BOUNDS)
                w00 = dhp * dwp
                w10 = dhp - w00
                w01 = dwp - w00
                w11 = 1.0 - dhp - dwp + w00
                for k in range(C // LANES):
                    cs = pl.ds(k * LANES, LANES)
                    acc = (bufs[0][p, cs] * w00 + bufs[1][p, cs] * w10
                           + bufs[2][p, cs] * w01 + bufs[3][p, cs] * w11)
                    ov[p, cs] = acc
            return carry2

        lax.fori_loop(0, SUB // LANES, blend, 0)
        pltpu.sync_copy(ov, out.at[pl.ds(base + g * SUB, SUB)])

    fire_chunk(0, iA, (wa0, wa1), bufsA, semIA, semGA)

    def pair(i, carry):
        g0 = 2 * i
        g1 = g0 + 1

        @pl.when(g1 < NSUB)
        def _():
            fire_chunk(g1, iB, (wb0, wb1), bufsB, semIB, semGB)

        wait_chunk(bufsA, semGA)
        blend_write(g0, bufsA, (wa0, wa1))

        @pl.when(g1 < NSUB)
        def _():
            @pl.when(g1 + 1 < NSUB)
            def _():
                fire_chunk(g1 + 1, iA, (wa0, wa1), bufsA, semIA, semGA)

            wait_chunk(bufsB, semGB)
            blend_write(g1, bufsB, (wb0, wb1))

        return carry

    lax.fori_loop(0, (NSUB + 1) // 2, pair, 0)


def kernel(input1, input2):
    padded = pl.pallas_call(
        _pad_body,
        grid=(B, C // 8),
        in_specs=[pl.BlockSpec((1, 8, H, W), lambda b, c: (b, c, 0, 0))],
        out_specs=pl.BlockSpec((1, 8, Hp, Wp), lambda b, c: (b, c, 0, 0)),
        out_shape=jax.ShapeDtypeStruct((B, C, Hp, Wp), jnp.float32),
    )(input1)
    tbl = padded.reshape(NROWS, C)

    RB = 8
    hw_spec = pl.BlockSpec((1, RB, W), lambda b, r: (b, r, 0))
    i00, i10, i01, i11, dhw, dww = pl.pallas_call(
        _prep_body,
        grid=(B, H // RB),
        in_specs=[pl.BlockSpec((1, 2, RB, W), lambda b, r: (b, 0, r, 0))],
        out_specs=[hw_spec] * 6,
        out_shape=[jax.ShapeDtypeStruct((B, H, W), jnp.int32)] * 4
        + [jax.ShapeDtypeStruct((B, H, W), jnp.float32)] * 2,
    )(input2)

    mesh = plsc.VectorSubcoreMesh(core_axis_name="c", subcore_axis_name="s")
    vals = functools.partial(
        pl.kernel,
        out_type=jax.ShapeDtypeStruct((BHW, C), jnp.float32),
        mesh=mesh,
        compiler_params=pltpu.CompilerParams(use_tc_tiling_on_sc=False),
        scratch_types=([pltpu.VMEM((SUB,), jnp.int32)] * 4
                       + [pltpu.VMEM((SUB,), jnp.float32)] * 2) * 2
        + [pltpu.VMEM((SUB, C), jnp.float32)] * 9
        + [pltpu.SemaphoreType.DMA] * 4,
    )(_sc_warp_body)(
        tbl,
        i00.reshape(BHW),
        i10.reshape(BHW),
        i01.reshape(BHW),
        i11.reshape(BHW),
        dhw.reshape(BHW),
        dww.reshape(BHW),
    )

    return vals.reshape(B, H, W, C).transpose(0, 3, 1, 2)
